# baseline (speedup 1.0000x reference)
import functools

import jax
import jax.numpy as jnp
from jax import lax
from jax.experimental import pallas as pl
from jax.experimental.pallas import tpu as pltpu
from jax.experimental.pallas import tpu_sc as plsc

B = 16384
D = 64
HALF = B // 2
MARGIN = 1.0

_info = plsc.get_sparse_core_info()
NC, NS, L = _info.num_cores, _info.num_subcores, _info.num_lanes
NW = NC * NS
PPW = HALF // NW
CH = 128
NCH = (2 * PPW) // CH
GRP = CH // 16


def _sc_body(bh, bt, br, ent_re, ent_im, rel_re, rel_im,
             score_out, losspart_out,
             idx_h, idx_t, idx_r,
             hre, him, tre, tim, rre, rim,
             score_v, loss_v, isem, gsem):
    w = lax.axis_index("s") * NC + lax.axis_index("c")
    pos_base = w * PPW
    neg_base = HALF + w * PPW

    bases = [pos_base + c * CH if c < NCH // 2 else neg_base + (c - NCH // 2) * CH
             for c in range(NCH)]

    copies = []
    for c in range(NCH):
        copies.append(pltpu.async_copy(bh.at[pl.ds(bases[c], CH)], idx_h.at[c], isem))
        copies.append(pltpu.async_copy(bt.at[pl.ds(bases[c], CH)], idx_t.at[c], isem))
        copies.append(pltpu.async_copy(br.at[pl.ds(bases[c], CH)], idx_r.at[c], isem))
    for cp in copies:
        cp.wait()

    lane = lax.iota(jnp.int32, L)
    shuf = [lane ^ sh for sh in (8, 4, 2, 1)]

    def hsum(v):
        for idx in shuf:
            v = v + v.at[idx].get(mode="promise_in_bounds")
        return v

    for c in range(NCH):
        gathers = [
            pltpu.async_copy(ent_re.at[idx_h.at[c]], hre, gsem),
            pltpu.async_copy(ent_im.at[idx_h.at[c]], him, gsem),
            pltpu.async_copy(ent_re.at[idx_t.at[c]], tre, gsem),
            pltpu.async_copy(ent_im.at[idx_t.at[c]], tim, gsem),
            pltpu.async_copy(rel_re.at[idx_r.at[c]], rre, gsem),
            pltpu.async_copy(rel_im.at[idx_r.at[c]], rim, gsem),
        ]
        for g in gathers:
            g.wait()

        def group_body(g, carry, c=c):
            scores = jnp.zeros((L,), jnp.float32)
            for e in range(16):
                row = g * 16 + e
                acc = jnp.zeros((L,), jnp.float32)
                for q in range(D // L):
                    sl = pl.ds(q * L, L)
                    a = hre[row, sl]
                    b = him[row, sl]
                    tr = tre[row, sl]
                    ti = tim[row, sl]
                    rr = rre[row, sl]
                    ri = rim[row, sl]
                    acc = acc + (a * tr + b * ti) * rr + (a * ti - b * tr) * ri
                scores = jnp.where(lane == e, hsum(acc), scores)
            score_v[pl.ds(c * CH + g * 16, L)] = scores
            return carry

        lax.fori_loop(0, GRP, group_body, 0)

    pltpu.sync_copy(score_v.at[pl.ds(0, PPW)], score_out.at[pl.ds(pos_base, PPW)])
    pltpu.sync_copy(score_v.at[pl.ds(PPW, PPW)], score_out.at[pl.ds(neg_base, PPW)])

    acc = jnp.zeros((L,), jnp.float32)
    for j in range(PPW // L):
        p = score_v[pl.ds(j * L, L)]
        n = score_v[pl.ds(PPW + j * L, L)]
        acc = acc + jnp.maximum(0.0, p - n + MARGIN)
    loss_v[...] = acc
    pltpu.sync_copy(loss_v, losspart_out.at[w])


@functools.partial(
    pl.kernel,
    mesh=plsc.VectorSubcoreMesh(core_axis_name="c", subcore_axis_name="s"),
    compiler_params=pltpu.CompilerParams(use_tc_tiling_on_sc=False),
    out_type=[
        jax.ShapeDtypeStruct((B,), jnp.float32),
        jax.ShapeDtypeStruct((NW, L), jnp.float32),
    ],
    scratch_types=[
        pltpu.VMEM((NCH, CH), jnp.int32),
        pltpu.VMEM((NCH, CH), jnp.int32),
        pltpu.VMEM((NCH, CH), jnp.int32),
        pltpu.VMEM((CH, D), jnp.float32),
        pltpu.VMEM((CH, D), jnp.float32),
        pltpu.VMEM((CH, D), jnp.float32),
        pltpu.VMEM((CH, D), jnp.float32),
        pltpu.VMEM((CH, D), jnp.float32),
        pltpu.VMEM((CH, D), jnp.float32),
        pltpu.VMEM((2 * PPW,), jnp.float32),
        pltpu.VMEM((L,), jnp.float32),
        pltpu.SemaphoreType.DMA,
        pltpu.SemaphoreType.DMA,
    ],
)
def _sc_kernel(bh, bt, br, ent_re, ent_im, rel_re, rel_im, *rest):
    _sc_body(bh, bt, br, ent_re, ent_im, rel_re, rel_im, *rest)


def kernel(batch_h, batch_t, batch_r, batch_y, ent_re, ent_im, rel_re, rel_im):
    bh = batch_h.astype(jnp.int32)
    bt = batch_t.astype(jnp.int32)
    br = batch_r.astype(jnp.int32)
    score, losspart = _sc_kernel(bh, bt, br, ent_re, ent_im, rel_re, rel_im)
    loss = jnp.sum(losspart)
    return (loss, score[:HALF], score[HALF:], score)
